# Initial kernel scaffold; baseline (speedup 1.0000x reference)
import jax
import jax.numpy as jnp
from jax.experimental import pallas as pl


def kernel(W, slope, power, w_ini, timesteps):
    raise NotImplementedError("write your pallas kernel here")


# speedup vs baseline: 4.3982x; 4.3982x over previous
import functools

import jax
import jax.numpy as jnp
from jax import lax
from jax.experimental import pallas as pl
from jax.experimental.pallas import tpu as pltpu
from jax.experimental.pallas import tpu_sc as plsc

NUM_TIMESTEPS = 1000
BATCH = 16384
L = 16
NC, NS = 2, 16
NW = NC * NS
BPW = BATCH // NW
NT_PAD = 1008
NCHUNK = NT_PAD // L


def _snr_body(par_hbm, w_hbm, ts_hbm, out_hbm, par_v, w_v, tab_v, ts_v, out_v):
    wid = lax.axis_index("s") * NC + lax.axis_index("c")
    base = wid * BPW

    pltpu.sync_copy(ts_hbm.at[pl.ds(base, BPW)], ts_v)
    pltpu.sync_copy(par_hbm, par_v)
    pltpu.sync_copy(w_hbm, w_v)

    wini_vec = par_v[pl.ds(0, L)]
    slope_vec = par_v[pl.ds(L, L)]

    carry0 = 0.0 - slope_vec

    def build(j, carry):
        v = jnp.maximum(w_v[pl.ds(j * L, L)] + wini_vec, 0.0)
        s = plsc.cumsum(v) + carry
        tab_v[pl.ds(j * L, L)] = s
        last = jnp.full((L,), L - 1, jnp.int32) + j * L
        return plsc.load_gather(tab_v, [last])

    lax.fori_loop(0, NCHUNK, build, carry0)

    neg_inf = jnp.full((L,), -jnp.inf, jnp.float32)
    for i in range(BPW // L):
        t = ts_v[pl.ds(i * L, L)]
        idx = jnp.maximum(t - 1, 0)
        val = plsc.load_gather(tab_v, [idx])
        out_v[pl.ds(i * L, L)] = jnp.where(t == 0, neg_inf, val)

    pltpu.sync_copy(out_v, out_hbm.at[pl.ds(base, BPW)])


@jax.jit
def kernel(W, slope, power, w_ini, timesteps):
    del power
    w_pad = jnp.concatenate([W.astype(jnp.float32),
                             jnp.zeros((NT_PAD - NUM_TIMESTEPS,), jnp.float32)])
    params = jnp.concatenate([
        jnp.full((L,), w_ini, jnp.float32),
        jnp.broadcast_to(slope.astype(jnp.float32), (L,)),
    ])
    run = pl.kernel(
        _snr_body,
        out_type=jax.ShapeDtypeStruct((BATCH,), jnp.float32),
        mesh=plsc.VectorSubcoreMesh(core_axis_name="c", subcore_axis_name="s"),
        compiler_params=pltpu.CompilerParams(needs_layout_passes=False),
        scratch_types=[
            pltpu.VMEM((2 * L,), jnp.float32),
            pltpu.VMEM((NT_PAD,), jnp.float32),
            pltpu.VMEM((NT_PAD,), jnp.float32),
            pltpu.VMEM((BPW,), jnp.int32),
            pltpu.VMEM((BPW,), jnp.float32),
        ],
    )
    return run(params, w_pad, timesteps)
